# baseline (speedup 1.0000x reference)
import functools

import jax
import jax.numpy as jnp
from jax import lax
from jax.experimental import pallas as pl
from jax.experimental.pallas import tpu as pltpu
from jax.experimental.pallas import tpu_sc as plsc

_NC = 2
_NS = 16
_NW = _NC * _NS

_CH = 128


def _transpose_body(w_ref, out_ref):
    out_ref[...] = w_ref[...].T


def _transpose(W_E, blk):
    D, V = W_E.shape
    return pl.pallas_call(
        _transpose_body,
        grid=(pl.cdiv(V, blk),),
        in_specs=[pl.BlockSpec((D, blk), lambda i: (0, i))],
        out_specs=pl.BlockSpec((blk, D), lambda i: (i, 0)),
        out_shape=jax.ShapeDtypeStruct((V, D), W_E.dtype),
    )(W_E)


def _gather_body(n_ch, table_hbm, idx_hbm, out_hbm, idx_v, rows_v, sem):
    wid = lax.axis_index("s") * _NC + lax.axis_index("c")
    pltpu.sync_copy(idx_hbm.at[wid], idx_v)
    base = wid * (n_ch * _CH)

    def body(ch, carry):
        pltpu.async_copy(table_hbm.at[idx_v.at[ch]], rows_v, sem).wait()
        pltpu.sync_copy(rows_v, out_hbm.at[pl.ds(base + ch * _CH, _CH)])
        return carry

    lax.fori_loop(0, n_ch, body, 0)


def _gather(W_T, idx3):
    nw, n_ch, ch = idx3.shape
    V, D = W_T.shape
    B = nw * n_ch * ch

    mesh = plsc.VectorSubcoreMesh(core_axis_name="c", subcore_axis_name="s")
    f = pl.kernel(
        functools.partial(_gather_body, n_ch),
        out_type=jax.ShapeDtypeStruct((B, D), jnp.float32),
        mesh=mesh,
        scratch_types=[
            pltpu.VMEM((n_ch, ch), jnp.int32),
            pltpu.VMEM((ch, D), jnp.float32),
            pltpu.SemaphoreType.DMA,
        ],
        compiler_params=pltpu.CompilerParams(use_tc_tiling_on_sc=False),
    )
    return f(W_T, idx3)


def kernel(x, W_E):
    D, V = W_E.shape
    B, P = x.shape
    W_T = _transpose(W_E, 8192)
    idx3 = x.reshape(_NW, (B * P) // (_NW * _CH), _CH).astype(jnp.int32)
    out = _gather(W_T, idx3)
    return out.reshape(B, P, D)


# speedup vs baseline: 1.0871x; 1.0871x over previous
import functools

import jax
import jax.numpy as jnp
from jax import lax
from jax.experimental import pallas as pl
from jax.experimental.pallas import tpu as pltpu
from jax.experimental.pallas import tpu_sc as plsc

_NC = 2
_NS = 16
_NW = _NC * _NS

_CH = 128


def _gather_body(n_ch, table_hbm, idx_hbm, out_hbm, idx_v, rows_v, gsem0,
                 gsem1):
    wid = lax.axis_index("s") * _NC + lax.axis_index("c")
    pltpu.sync_copy(idx_hbm.at[wid], idx_v)
    base = wid * (n_ch * _CH)

    pltpu.async_copy(table_hbm.at[idx_v.at[0]], rows_v.at[0], gsem0)

    def pair_body(i, carry):
        for par, my_sem, other_sem in ((0, gsem0, gsem1), (1, gsem1, gsem0)):
            ch = 2 * i + par
            cur = rows_v.at[par]
            nxt = rows_v.at[1 - par]
            pltpu.make_async_copy(table_hbm.at[idx_v.at[ch]], cur,
                                  my_sem).wait()

            @pl.when(ch + 1 < n_ch)
            def _():
                pltpu.async_copy(table_hbm.at[idx_v.at[ch + 1]], nxt,
                                 other_sem)

            pltpu.sync_copy(cur, out_hbm.at[pl.ds(base + ch * _CH, _CH)])
        return carry

    lax.fori_loop(0, n_ch // 2, pair_body, 0)


def _gather(W_T, idx3):
    nw, n_ch, ch = idx3.shape
    V, D = W_T.shape
    B = nw * n_ch * ch

    mesh = plsc.VectorSubcoreMesh(core_axis_name="c", subcore_axis_name="s")
    f = pl.kernel(
        functools.partial(_gather_body, n_ch),
        out_type=jax.ShapeDtypeStruct((B, D), jnp.float32),
        mesh=mesh,
        scratch_types=[
            pltpu.VMEM((n_ch, ch), jnp.int32),
            pltpu.VMEM((2, ch, D), jnp.float32),
            pltpu.SemaphoreType.DMA,
            pltpu.SemaphoreType.DMA,
        ],
        compiler_params=pltpu.CompilerParams(use_tc_tiling_on_sc=False),
    )
    return f(W_T, idx3)


def kernel(x, W_E):
    D, V = W_E.shape
    B, P = x.shape
    W_T = jnp.swapaxes(W_E, 0, 1)
    idx3 = x.reshape(_NW, (B * P) // (_NW * _CH), _CH).astype(jnp.int32)
    out = _gather(W_T, idx3)
    return out.reshape(B, P, D)


# speedup vs baseline: 1.3252x; 1.2189x over previous
import functools

import jax
import jax.numpy as jnp
from jax import lax
from jax.experimental import pallas as pl
from jax.experimental.pallas import tpu as pltpu
from jax.experimental.pallas import tpu_sc as plsc

_NC = 2
_NS = 16
_NW = _NC * _NS

_CH = 128


def _gather_body(n_ch, D, table_hbm, idx_hbm, out_hbm, idx_v, rows_v, gsem0,
                 gsem1):
    wid = lax.axis_index("s") * _NC + lax.axis_index("c")
    pltpu.sync_copy(idx_hbm.at[wid], idx_v)
    base = wid * (n_ch * _CH)

    pltpu.async_copy(table_hbm.at[idx_v.at[0]], rows_v.at[0], gsem0)

    def pair_body(i, carry):
        for par, my_sem, other_sem in ((0, gsem0, gsem1), (1, gsem1, gsem0)):
            ch = 2 * i + par
            cur = rows_v.at[par]
            nxt = rows_v.at[1 - par]
            pltpu.make_async_copy(table_hbm.at[idx_v.at[ch]], cur,
                                  my_sem).wait()

            @pl.when(ch + 1 < n_ch)
            def _():
                pltpu.async_copy(table_hbm.at[idx_v.at[ch + 1]], nxt,
                                 other_sem)

            pltpu.sync_copy(cur, out_hbm.at[pl.ds(base + ch * _CH, _CH)])
        return carry

    lax.fori_loop(0, n_ch // 2, pair_body, 0)


def _gather(W_pad, idx3, D):
    nw, n_ch, ch = idx3.shape
    V, R = W_pad.shape
    B = nw * n_ch * ch

    mesh = plsc.VectorSubcoreMesh(core_axis_name="c", subcore_axis_name="s")
    f = pl.kernel(
        functools.partial(_gather_body, n_ch, D),
        out_type=jax.ShapeDtypeStruct((B, R), jnp.float32),
        mesh=mesh,
        scratch_types=[
            pltpu.VMEM((n_ch, ch), jnp.int32),
            pltpu.VMEM((2, ch, R), jnp.float32),
            pltpu.SemaphoreType.DMA,
            pltpu.SemaphoreType.DMA,
        ],
        compiler_params=pltpu.CompilerParams(use_tc_tiling_on_sc=True),
    )
    return f(W_pad, idx3)


def kernel(x, W_E):
    D, V = W_E.shape
    B, P = x.shape
    W_pad = jnp.pad(jnp.swapaxes(W_E, 0, 1), ((0, 0), (0, 128 - D)))
    idx3 = x.reshape(_NW, (B * P) // (_NW * _CH), _CH).astype(jnp.int32)
    out = _gather(W_pad, idx3, D)
    return out[:, :D].reshape(B, P, D)


# speedup vs baseline: 1.4234x; 1.0742x over previous
import functools

import jax
import jax.numpy as jnp
from jax import lax
from jax.experimental import pallas as pl
from jax.experimental.pallas import tpu as pltpu
from jax.experimental.pallas import tpu_sc as plsc

_NC = 2
_NS = 16
_NW = _NC * _NS

_CH = 128


_NBUF = 5
_DEPTH = 3


def _gather_body(n_ch, D, table_hbm, idx_hbm, out_hbm, idx_v, rows_v, gsems,
                 ssems):
    wid = lax.axis_index("s") * _NC + lax.axis_index("c")
    pltpu.sync_copy(idx_hbm.at[wid], idx_v)
    base = wid * (n_ch * _CH)

    def gather(ch, b):
        return pltpu.make_async_copy(table_hbm.at[idx_v.at[ch]],
                                     rows_v.at[b], gsems.at[b])

    def store(ch, b):
        return pltpu.make_async_copy(
            rows_v.at[b], out_hbm.at[pl.ds(base + ch * _CH, _CH)],
            ssems.at[b])

    for j in range(_DEPTH):
        gather(j, j).start()

    def ring_body(i, carry):
        for k in range(_NBUF):
            j = _NBUF * i + k
            gather(j, k).wait()
            store(j, k).start()

            @pl.when(j >= 2)
            def _():
                store(j - 2, (k - 2) % _NBUF).wait()

            @pl.when(j + _DEPTH < n_ch)
            def _():
                gather(j + _DEPTH, (k + _DEPTH) % _NBUF).start()
        return carry

    lax.fori_loop(0, n_ch // _NBUF, ring_body, 0)

    store(n_ch - 2, (n_ch - 2) % _NBUF).wait()
    store(n_ch - 1, (n_ch - 1) % _NBUF).wait()


def _gather(W_pad, idx3, D):
    nw, n_ch, ch = idx3.shape
    V, R = W_pad.shape
    B = nw * n_ch * ch

    mesh = plsc.VectorSubcoreMesh(core_axis_name="c", subcore_axis_name="s")
    f = pl.kernel(
        functools.partial(_gather_body, n_ch, D),
        out_type=jax.ShapeDtypeStruct((B, R), jnp.float32),
        mesh=mesh,
        scratch_types=[
            pltpu.VMEM((n_ch, ch), jnp.int32),
            pltpu.VMEM((_NBUF, ch, R), jnp.float32),
            pltpu.SemaphoreType.DMA((_NBUF,)),
            pltpu.SemaphoreType.DMA((_NBUF,)),
        ],
        compiler_params=pltpu.CompilerParams(use_tc_tiling_on_sc=True),
    )
    return f(W_pad, idx3)


def kernel(x, W_E):
    D, V = W_E.shape
    B, P = x.shape
    W_pad = jnp.pad(jnp.swapaxes(W_E, 0, 1), ((0, 0), (0, 128 - D)))
    idx3 = x.reshape(_NW, (B * P) // (_NW * _CH), _CH).astype(jnp.int32)
    out = _gather(W_pad, idx3, D)
    return out[:, :D].reshape(B, P, D)


# speedup vs baseline: 1.5083x; 1.0596x over previous
import functools

import jax
import jax.numpy as jnp
from jax import lax
from jax.experimental import pallas as pl
from jax.experimental.pallas import tpu as pltpu
from jax.experimental.pallas import tpu_sc as plsc

_NC = 2
_NS = 16
_NW = _NC * _NS

_CH = 128


_TBLK = 2048


def _transpose_pad_body(w_ref, out_ref):
    out_ref[:, 0:64] = w_ref[...].T
    out_ref[:, 64:128] = jnp.zeros((_TBLK, 64), jnp.float32)


def _transpose_pad(W_E):
    D, V = W_E.shape
    return pl.pallas_call(
        _transpose_pad_body,
        grid=(pl.cdiv(V, _TBLK),),
        in_specs=[pl.BlockSpec((D, _TBLK), lambda i: (0, i))],
        out_specs=pl.BlockSpec((_TBLK, 128), lambda i: (i, 0)),
        out_shape=jax.ShapeDtypeStruct((V, 128), jnp.float32),
    )(W_E)


_NBUF = 5
_DEPTH = 3


def _gather_body(n_ch, D, table_hbm, idx_hbm, out_hbm, idx_v, rows_v, gsems,
                 ssems):
    wid = lax.axis_index("s") * _NC + lax.axis_index("c")
    pltpu.sync_copy(idx_hbm.at[wid], idx_v)
    base = wid * (n_ch * _CH)

    def gather(ch, b):
        return pltpu.make_async_copy(table_hbm.at[idx_v.at[ch]],
                                     rows_v.at[b], gsems.at[b])

    def store(ch, b):
        return pltpu.make_async_copy(
            rows_v.at[b], out_hbm.at[pl.ds(base + ch * _CH, _CH)],
            ssems.at[b])

    for j in range(_DEPTH):
        gather(j, j).start()

    def ring_body(i, carry):
        for k in range(_NBUF):
            j = _NBUF * i + k

            gather(j, k).wait()
            store(j, k).start()

            @pl.when(j >= 2)
            def _():
                store(j - 2, (k - 2) % _NBUF).wait()

            @pl.when(j + _DEPTH < n_ch)
            def _():
                gather(j + _DEPTH, (k + _DEPTH) % _NBUF).start()
        return carry

    lax.fori_loop(0, n_ch // _NBUF, ring_body, 0)

    store(n_ch - 2, (n_ch - 2) % _NBUF).wait()
    store(n_ch - 1, (n_ch - 1) % _NBUF).wait()


def _gather(W_pad, idx3, D):
    nw, n_ch, ch = idx3.shape
    V, R = W_pad.shape
    B = nw * n_ch * ch

    mesh = plsc.VectorSubcoreMesh(core_axis_name="c", subcore_axis_name="s")
    f = pl.kernel(
        functools.partial(_gather_body, n_ch, D),
        out_type=jax.ShapeDtypeStruct((B, R), jnp.float32),
        mesh=mesh,
        scratch_types=[
            pltpu.VMEM((n_ch, ch), jnp.int32),
            pltpu.VMEM((_NBUF, ch, R), jnp.float32),
            pltpu.SemaphoreType.DMA((_NBUF,)),
            pltpu.SemaphoreType.DMA((_NBUF,)),
        ],
        compiler_params=pltpu.CompilerParams(use_tc_tiling_on_sc=True),
    )
    return f(W_pad, idx3)


def kernel(x, W_E):
    D, V = W_E.shape
    B, P = x.shape
    W_pad = _transpose_pad(W_E)
    idx3 = x.reshape(_NW, (B * P) // (_NW * _CH), _CH).astype(jnp.int32)
    out = _gather(W_pad, idx3, D)
    return out[:, :D].reshape(B, P, D)


# speedup vs baseline: 1.5166x; 1.0055x over previous
import functools

import jax
import jax.numpy as jnp
from jax import lax
from jax.experimental import pallas as pl
from jax.experimental.pallas import tpu as pltpu
from jax.experimental.pallas import tpu_sc as plsc

_NC = 2
_NS = 16
_NW = _NC * _NS

_CH = 128


_TBLK = 2048


def _transpose_pad_body(w_ref, out_ref):
    out_ref[:, 0:64] = w_ref[...].T


def _transpose_pad(W_E):
    D, V = W_E.shape
    return pl.pallas_call(
        _transpose_pad_body,
        grid=(pl.cdiv(V, _TBLK),),
        in_specs=[pl.BlockSpec((D, _TBLK), lambda i: (0, i))],
        out_specs=pl.BlockSpec((_TBLK, 128), lambda i: (i, 0)),
        out_shape=jax.ShapeDtypeStruct((V, 128), jnp.float32),
    )(W_E)


_NBUF = 5
_DEPTH = 3


def _gather_body(n_ch, D, table_hbm, idx_hbm, out_hbm, idx_v, rows_v, gsems,
                 ssems):
    wid = lax.axis_index("s") * _NC + lax.axis_index("c")
    pltpu.sync_copy(idx_hbm.at[wid], idx_v)
    base = wid * (n_ch * _CH)

    def gather(ch, b):
        return pltpu.make_async_copy(table_hbm.at[idx_v.at[ch]],
                                     rows_v.at[b], gsems.at[b])

    def store(ch, b):
        return pltpu.make_async_copy(
            rows_v.at[b], out_hbm.at[pl.ds(base + ch * _CH, _CH)],
            ssems.at[b])

    for j in range(_DEPTH):
        gather(j, j).start()

    def ring_body(i, carry):
        for k in range(_NBUF):
            j = _NBUF * i + k

            gather(j, k).wait()
            store(j, k).start()

            @pl.when(j >= 2)
            def _():
                store(j - 2, (k - 2) % _NBUF).wait()

            @pl.when(j + _DEPTH < n_ch)
            def _():
                gather(j + _DEPTH, (k + _DEPTH) % _NBUF).start()
        return carry

    lax.fori_loop(0, n_ch // _NBUF, ring_body, 0)

    store(n_ch - 2, (n_ch - 2) % _NBUF).wait()
    store(n_ch - 1, (n_ch - 1) % _NBUF).wait()


def _gather(W_pad, idx3, D):
    nw, n_ch, ch = idx3.shape
    V, R = W_pad.shape
    B = nw * n_ch * ch

    mesh = plsc.VectorSubcoreMesh(core_axis_name="c", subcore_axis_name="s")
    f = pl.kernel(
        functools.partial(_gather_body, n_ch, D),
        out_type=jax.ShapeDtypeStruct((B, R), jnp.float32),
        mesh=mesh,
        scratch_types=[
            pltpu.VMEM((n_ch, ch), jnp.int32),
            pltpu.VMEM((_NBUF, ch, R), jnp.float32),
            pltpu.SemaphoreType.DMA((_NBUF,)),
            pltpu.SemaphoreType.DMA((_NBUF,)),
        ],
        compiler_params=pltpu.CompilerParams(use_tc_tiling_on_sc=True),
    )
    return f(W_pad, idx3)


def kernel(x, W_E):
    D, V = W_E.shape
    B, P = x.shape
    W_pad = _transpose_pad(W_E)
    idx3 = x.reshape(_NW, (B * P) // (_NW * _CH), _CH).astype(jnp.int32)
    out = _gather(W_pad, idx3, D)
    return out[:, :D].reshape(B, P, D)


# speedup vs baseline: 1.9077x; 1.2579x over previous
import functools

import jax
import jax.numpy as jnp
from jax import lax
from jax.experimental import pallas as pl
from jax.experimental.pallas import tpu as pltpu
from jax.experimental.pallas import tpu_sc as plsc

_NC = 2
_NS = 16
_NW = _NC * _NS

_CH = 128


_TBLK = 8192


def _transpose_pad_body(w_ref, out_ref):
    out_ref[:, 0:64] = w_ref[...].T


def _transpose_pad(W_E):
    D, V = W_E.shape
    return pl.pallas_call(
        _transpose_pad_body,
        grid=(pl.cdiv(V, _TBLK),),
        in_specs=[pl.BlockSpec((D, _TBLK), lambda i: (0, i))],
        out_specs=pl.BlockSpec((_TBLK, 128), lambda i: (i, 0)),
        out_shape=jax.ShapeDtypeStruct((V, 128), jnp.float32),
    )(W_E)


_NBUF = 5
_DEPTH = 3


def _gather_body(n_ch, D, table_hbm, idx_hbm, out_hbm, idx_v, rows_v, gsems,
                 ssems):
    wid = lax.axis_index("s") * _NC + lax.axis_index("c")
    pltpu.sync_copy(idx_hbm.at[wid], idx_v)
    base = wid * (n_ch * _CH)

    def gather(ch, b):
        return pltpu.make_async_copy(table_hbm.at[idx_v.at[ch]],
                                     rows_v.at[b], gsems.at[b])

    def store(ch, b):
        return pltpu.make_async_copy(
            rows_v.at[b], out_hbm.at[pl.ds(base + ch * _CH, _CH)],
            ssems.at[b])

    for j in range(_DEPTH):
        gather(j, j).start()

    def ring_body(i, carry):
        for k in range(_NBUF):
            j = _NBUF * i + k

            gather(j, k).wait()
            store(j, k).start()

            @pl.when(j >= 2)
            def _():
                store(j - 2, (k - 2) % _NBUF).wait()

            @pl.when(j + _DEPTH < n_ch)
            def _():
                gather(j + _DEPTH, (k + _DEPTH) % _NBUF).start()
        return carry

    lax.fori_loop(0, n_ch // _NBUF, ring_body, 0)

    store(n_ch - 2, (n_ch - 2) % _NBUF).wait()
    store(n_ch - 1, (n_ch - 1) % _NBUF).wait()


def _gather(W_pad, idx3, D):
    nw, n_ch, ch = idx3.shape
    V, R = W_pad.shape
    B = nw * n_ch * ch

    mesh = plsc.VectorSubcoreMesh(core_axis_name="c", subcore_axis_name="s")
    f = pl.kernel(
        functools.partial(_gather_body, n_ch, D),
        out_type=jax.ShapeDtypeStruct((B, R), jnp.float32),
        mesh=mesh,
        scratch_types=[
            pltpu.VMEM((n_ch, ch), jnp.int32),
            pltpu.VMEM((_NBUF, ch, R), jnp.float32),
            pltpu.SemaphoreType.DMA((_NBUF,)),
            pltpu.SemaphoreType.DMA((_NBUF,)),
        ],
        compiler_params=pltpu.CompilerParams(use_tc_tiling_on_sc=True),
    )
    return f(W_pad, idx3)


def kernel(x, W_E):
    D, V = W_E.shape
    B, P = x.shape
    W_pad = _transpose_pad(W_E)
    idx3 = x.reshape(_NW, (B * P) // (_NW * _CH), _CH).astype(jnp.int32)
    out = _gather(W_pad, idx3, D)
    return out[:, :D].reshape(B, P, D)


# speedup vs baseline: 1.9542x; 1.0244x over previous
import functools

import jax
import jax.numpy as jnp
from jax import lax
from jax.experimental import pallas as pl
from jax.experimental.pallas import tpu as pltpu
from jax.experimental.pallas import tpu_sc as plsc

_NC = 2
_NS = 16
_NW = _NC * _NS

_CH = 128


_TBLK = 16384


def _transpose_pad_body(w_ref, out_ref):
    out_ref[:, 0:64] = w_ref[...].T


def _transpose_pad(W_E):
    D, V = W_E.shape
    return pl.pallas_call(
        _transpose_pad_body,
        grid=(pl.cdiv(V, _TBLK),),
        in_specs=[pl.BlockSpec((D, _TBLK), lambda i: (0, i))],
        out_specs=pl.BlockSpec((_TBLK, 128), lambda i: (i, 0)),
        out_shape=jax.ShapeDtypeStruct((V, 128), jnp.float32),
    )(W_E)


_NBUF = 5
_DEPTH = 3


def _gather_body(n_ch, D, table_hbm, idx_hbm, out_hbm, idx_v, rows_v, gsems,
                 ssems):
    wid = lax.axis_index("s") * _NC + lax.axis_index("c")
    pltpu.sync_copy(idx_hbm.at[wid], idx_v)
    base = wid * (n_ch * _CH)

    def gather(ch, b):
        return pltpu.make_async_copy(table_hbm.at[idx_v.at[ch]],
                                     rows_v.at[b], gsems.at[b])

    def store(ch, b):
        return pltpu.make_async_copy(
            rows_v.at[b], out_hbm.at[pl.ds(base + ch * _CH, _CH)],
            ssems.at[b])

    for j in range(_DEPTH):
        gather(j, j).start()

    def ring_body(i, carry):
        for k in range(_NBUF):
            j = _NBUF * i + k

            gather(j, k).wait()
            store(j, k).start()

            @pl.when(j >= 2)
            def _():
                store(j - 2, (k - 2) % _NBUF).wait()

            @pl.when(j + _DEPTH < n_ch)
            def _():
                gather(j + _DEPTH, (k + _DEPTH) % _NBUF).start()
        return carry

    lax.fori_loop(0, n_ch // _NBUF, ring_body, 0)

    store(n_ch - 2, (n_ch - 2) % _NBUF).wait()
    store(n_ch - 1, (n_ch - 1) % _NBUF).wait()


def _gather(W_pad, idx3, D):
    nw, n_ch, ch = idx3.shape
    V, R = W_pad.shape
    B = nw * n_ch * ch

    mesh = plsc.VectorSubcoreMesh(core_axis_name="c", subcore_axis_name="s")
    f = pl.kernel(
        functools.partial(_gather_body, n_ch, D),
        out_type=jax.ShapeDtypeStruct((B, R), jnp.float32),
        mesh=mesh,
        scratch_types=[
            pltpu.VMEM((n_ch, ch), jnp.int32),
            pltpu.VMEM((_NBUF, ch, R), jnp.float32),
            pltpu.SemaphoreType.DMA((_NBUF,)),
            pltpu.SemaphoreType.DMA((_NBUF,)),
        ],
        compiler_params=pltpu.CompilerParams(use_tc_tiling_on_sc=True),
    )
    return f(W_pad, idx3)


def kernel(x, W_E):
    D, V = W_E.shape
    B, P = x.shape
    W_pad = _transpose_pad(W_E)
    idx3 = x.reshape(_NW, (B * P) // (_NW * _CH), _CH).astype(jnp.int32)
    out = _gather(W_pad, idx3, D)
    return out[:, :D].reshape(B, P, D)


# speedup vs baseline: 1.9812x; 1.0138x over previous
import functools

import jax
import jax.numpy as jnp
from jax import lax
from jax.experimental import pallas as pl
from jax.experimental.pallas import tpu as pltpu
from jax.experimental.pallas import tpu_sc as plsc

_NC = 2
_NS = 16
_NW = _NC * _NS

_CH = 128


_TBLK = 32768


def _transpose_pad_body(w_ref, out_ref):
    out_ref[:, 0:64] = w_ref[...].T


def _transpose_pad(W_E):
    D, V = W_E.shape
    return pl.pallas_call(
        _transpose_pad_body,
        grid=(pl.cdiv(V, _TBLK),),
        in_specs=[pl.BlockSpec((D, _TBLK), lambda i: (0, i))],
        out_specs=pl.BlockSpec((_TBLK, 128), lambda i: (i, 0)),
        out_shape=jax.ShapeDtypeStruct((V, 128), jnp.float32),
    )(W_E)


_NBUF = 5
_DEPTH = 3


def _gather_body(n_ch, D, table_hbm, idx_hbm, out_hbm, idx_v, rows_v, gsems,
                 ssems):
    wid = lax.axis_index("s") * _NC + lax.axis_index("c")
    pltpu.sync_copy(idx_hbm.at[wid], idx_v)
    base = wid * (n_ch * _CH)

    def gather(ch, b):
        return pltpu.make_async_copy(table_hbm.at[idx_v.at[ch]],
                                     rows_v.at[b], gsems.at[b])

    def store(ch, b):
        return pltpu.make_async_copy(
            rows_v.at[b], out_hbm.at[pl.ds(base + ch * _CH, _CH)],
            ssems.at[b])

    for j in range(_DEPTH):
        gather(j, j).start()

    def ring_body(i, carry):
        for k in range(_NBUF):
            j = _NBUF * i + k

            gather(j, k).wait()
            store(j, k).start()

            @pl.when(j >= 2)
            def _():
                store(j - 2, (k - 2) % _NBUF).wait()

            @pl.when(j + _DEPTH < n_ch)
            def _():
                gather(j + _DEPTH, (k + _DEPTH) % _NBUF).start()
        return carry

    lax.fori_loop(0, n_ch // _NBUF, ring_body, 0)

    store(n_ch - 2, (n_ch - 2) % _NBUF).wait()
    store(n_ch - 1, (n_ch - 1) % _NBUF).wait()


def _gather(W_pad, idx3, D):
    nw, n_ch, ch = idx3.shape
    V, R = W_pad.shape
    B = nw * n_ch * ch

    mesh = plsc.VectorSubcoreMesh(core_axis_name="c", subcore_axis_name="s")
    f = pl.kernel(
        functools.partial(_gather_body, n_ch, D),
        out_type=jax.ShapeDtypeStruct((B, R), jnp.float32),
        mesh=mesh,
        scratch_types=[
            pltpu.VMEM((n_ch, ch), jnp.int32),
            pltpu.VMEM((_NBUF, ch, R), jnp.float32),
            pltpu.SemaphoreType.DMA((_NBUF,)),
            pltpu.SemaphoreType.DMA((_NBUF,)),
        ],
        compiler_params=pltpu.CompilerParams(use_tc_tiling_on_sc=True),
    )
    return f(W_pad, idx3)


def kernel(x, W_E):
    D, V = W_E.shape
    B, P = x.shape
    W_pad = _transpose_pad(W_E)
    idx3 = x.reshape(_NW, (B * P) // (_NW * _CH), _CH).astype(jnp.int32)
    out = _gather(W_pad, idx3, D)
    return out[:, :D].reshape(B, P, D)
